# baseline (speedup 1.0000x reference)
import functools

import jax
import jax.numpy as jnp
from jax import lax
from jax.experimental import pallas as pl
from jax.experimental.pallas import tpu as pltpu
from jax.experimental.pallas import tpu_sc as plsc

_LANES = 16
_ROW = 56


def _make_sc_gather(B):
    NC, NS = 2, 16
    NW = NC * NS
    bpw = B // NW
    half = bpw // 2
    mesh = plsc.VectorSubcoreMesh(core_axis_name="c", subcore_axis_name="s",
                                  num_cores=NC, num_subcores=NS)

    @functools.partial(
        pl.kernel,
        out_type=[
            jax.ShapeDtypeStruct((B, _ROW), jnp.float32),
            jax.ShapeDtypeStruct((B, _ROW), jnp.float32),
        ],
        mesh=mesh,
        scratch_types=[
            pltpu.VMEM((bpw,), jnp.int32),
            pltpu.VMEM((bpw,), jnp.int32),
            pltpu.VMEM((half, _ROW), jnp.float32),
            pltpu.VMEM((half, _ROW), jnp.float32),
            pltpu.SemaphoreType.DMA,
            pltpu.SemaphoreType.DMA,
        ],
    )
    def sc_gather(uid, mid, Uf, Mf, u_out, m_out,
                  idx_u, idx_m, urows, mrows, sem_u, sem_m):
        wid = lax.axis_index("s") * NC + lax.axis_index("c")
        base = wid * bpw
        pltpu.sync_copy(uid.at[pl.ds(base, bpw)], idx_u)
        pltpu.sync_copy(mid.at[pl.ds(base, bpw)], idx_m)

        for h in range(2):
            off = h * half

            def fire(g, _):
                vu = idx_u[pl.ds(off + g * _LANES, _LANES)] * _ROW
                vm = idx_m[pl.ds(off + g * _LANES, _LANES)] * _ROW
                for j in range(_LANES):
                    i = g * _LANES + j
                    du = pl.multiple_of(vu[j], 8)
                    dm = pl.multiple_of(vm[j], 8)
                    pltpu.async_copy(Uf.at[pl.ds(du, _ROW)],
                                     urows.at[i], sem_u)
                    pltpu.async_copy(Mf.at[pl.ds(dm, _ROW)],
                                     mrows.at[i], sem_m)
                return 0

            lax.fori_loop(0, half // _LANES, fire, 0)

            def drain(i, _):
                pltpu.make_async_copy(Uf.at[pl.ds(0, _ROW)],
                                      urows.at[i], sem_u).wait()
                pltpu.make_async_copy(Mf.at[pl.ds(0, _ROW)],
                                      mrows.at[i], sem_m).wait()
                return 0

            lax.fori_loop(0, half, drain, 0)
            pltpu.sync_copy(urows, u_out.at[pl.ds(base + off, half)])
            pltpu.sync_copy(mrows, m_out.at[pl.ds(base + off, half)])

    return sc_gather


def _mlp_body(u, m, rid, kw, age, R, Wk, bk, W1u, W1m, W1r, W1k, w1a, b1, W2,
              b2, out):
    f32 = jnp.float32
    NR = R.shape[0]
    k = jnp.maximum(jnp.dot(kw[:], Wk[:], preferred_element_type=f32) + bk[:],
                    0.0)
    onehot = (rid[:] == lax.broadcasted_iota(jnp.int32, (1, NR), 1)).astype(f32)
    r = jnp.dot(onehot, R[:], preferred_element_type=f32)
    h = (jnp.dot(u[:], W1u[:], preferred_element_type=f32)
         + jnp.dot(m[:], W1m[:], preferred_element_type=f32)
         + jnp.dot(r, W1r[:], preferred_element_type=f32)
         + jnp.dot(k, W1k[:], preferred_element_type=f32)
         + age[:] * w1a[:]
         + b1[:])
    h = jnp.maximum(h, 0.0)
    o = jnp.dot(h, W2[:], preferred_element_type=f32) + b2[:]
    out[:] = 1.0 / (1.0 + jnp.exp(-o))


def _make_mlp(B, DR, NR, KW, H1, BLK):
    grid = (B // BLK,)
    row = lambda i: (i, 0)
    rep = lambda i: (0, 0)
    return pl.pallas_call(
        _mlp_body,
        grid=grid,
        in_specs=[
            pl.BlockSpec((BLK, _ROW), row),
            pl.BlockSpec((BLK, _ROW), row),
            pl.BlockSpec((BLK, 1), row),
            pl.BlockSpec((BLK, KW), row),
            pl.BlockSpec((BLK, 1), row),
            pl.BlockSpec((NR, DR), rep),
            pl.BlockSpec((KW, 32), rep),
            pl.BlockSpec((1, 32), rep),
            pl.BlockSpec((_ROW, H1), rep),
            pl.BlockSpec((_ROW, H1), rep),
            pl.BlockSpec((DR, H1), rep),
            pl.BlockSpec((32, H1), rep),
            pl.BlockSpec((1, H1), rep),
            pl.BlockSpec((1, H1), rep),
            pl.BlockSpec((H1, 1), rep),
            pl.BlockSpec((1, 1), rep),
        ],
        out_specs=pl.BlockSpec((BLK, 1), row),
        out_shape=jax.ShapeDtypeStruct((B, 1), jnp.float32),
    )


def kernel(user_id, movie_id, region_id, keywords, age, U, M, R, Wk, bk, W1,
           b1, W2, b2):
    B = user_id.shape[0]
    DU, DM, DR = U.shape[1], M.shape[1], R.shape[1]
    NR = R.shape[0]
    KW = keywords.shape[1]
    H1 = W1.shape[1]

    Uf = jnp.pad(U, ((0, 0), (0, _ROW - DU))).reshape(-1)
    Mf = jnp.pad(M, ((0, 0), (0, _ROW - DM))).reshape(-1)

    sc_gather = _make_sc_gather(B)
    u, m = sc_gather(user_id, movie_id, Uf, Mf)

    W1u = jnp.pad(W1[:DU], ((0, _ROW - DU), (0, 0)))
    W1m = jnp.pad(W1[DU:DU + DM], ((0, _ROW - DM), (0, 0)))

    mlp = _make_mlp(B, DR, NR, KW, H1, BLK=2048)
    out = mlp(u, m, region_id.reshape(B, 1), keywords, age.reshape(B, 1),
              R, Wk, bk.reshape(1, -1),
              W1u, W1m, W1[DU + DM:DU + DM + DR],
              W1[DU + DM + DR:DU + DM + DR + 32], W1[-1:],
              b1.reshape(1, -1), W2, b2.reshape(1, 1))
    return out.reshape(B)


# speedup vs baseline: 3.6156x; 3.6156x over previous
import functools

import jax
import jax.numpy as jnp
from jax import lax
from jax.experimental import pallas as pl
from jax.experimental.pallas import tpu as pltpu
from jax.experimental.pallas import tpu_sc as plsc

_LANES = 16


def _make_sc_gather(B, DU, DM):
    NC, NS = 2, 16
    NW = NC * NS
    bpw = B // NW
    half = bpw // 2
    mesh = plsc.VectorSubcoreMesh(core_axis_name="c", subcore_axis_name="s",
                                  num_cores=NC, num_subcores=NS)

    @functools.partial(
        pl.kernel,
        out_type=[
            jax.ShapeDtypeStruct((B, DU), jnp.float32),
            jax.ShapeDtypeStruct((B, DM), jnp.float32),
        ],
        mesh=mesh,
        scratch_types=[
            pltpu.VMEM((bpw,), jnp.int32),
            pltpu.VMEM((bpw,), jnp.int32),
            pltpu.VMEM((half, DU), jnp.float32),
            pltpu.VMEM((half, DM), jnp.float32),
            pltpu.SemaphoreType.DMA,
            pltpu.SemaphoreType.DMA,
        ],
    )
    def sc_gather(uid, mid, U, M, u_out, m_out,
                  idx_u, idx_m, urows, mrows, sem_u, sem_m):
        wid = lax.axis_index("s") * NC + lax.axis_index("c")
        base = wid * bpw
        pltpu.sync_copy(uid.at[pl.ds(base, bpw)], idx_u)
        pltpu.sync_copy(mid.at[pl.ds(base, bpw)], idx_m)

        for h in range(2):
            off = h * half

            def fire(g, _):
                vu = idx_u[pl.ds(off + g * _LANES, _LANES)]
                vm = idx_m[pl.ds(off + g * _LANES, _LANES)]
                for j in range(_LANES):
                    i = g * _LANES + j
                    pltpu.async_copy(U.at[vu[j]], urows.at[i], sem_u)
                    pltpu.async_copy(M.at[vm[j]], mrows.at[i], sem_m)
                return 0

            lax.fori_loop(0, half // _LANES, fire, 0)

            def drain(i, _):
                pltpu.make_async_copy(U.at[0], urows.at[i], sem_u).wait()
                pltpu.make_async_copy(M.at[0], mrows.at[i], sem_m).wait()
                return 0

            lax.fori_loop(0, half, drain, 0)
            pltpu.sync_copy(urows, u_out.at[pl.ds(base + off, half)])
            pltpu.sync_copy(mrows, m_out.at[pl.ds(base + off, half)])

    return sc_gather


def _mlp_body(u, m, rid, kw, age, R, Wk, bk, W1u, W1m, W1r, W1k, w1a, b1, W2,
              b2, out):
    f32 = jnp.float32
    NR = R.shape[0]
    k = jnp.maximum(jnp.dot(kw[:], Wk[:], preferred_element_type=f32) + bk[:],
                    0.0)
    onehot = (rid[:] == lax.broadcasted_iota(jnp.int32, (1, NR), 1)).astype(f32)
    r = jnp.dot(onehot, R[:], preferred_element_type=f32)
    h = (jnp.dot(u[:], W1u[:], preferred_element_type=f32)
         + jnp.dot(m[:], W1m[:], preferred_element_type=f32)
         + jnp.dot(r, W1r[:], preferred_element_type=f32)
         + jnp.dot(k, W1k[:], preferred_element_type=f32)
         + age[:] * w1a[:]
         + b1[:])
    h = jnp.maximum(h, 0.0)
    o = jnp.dot(h, W2[:], preferred_element_type=f32) + b2[:]
    out[:] = 1.0 / (1.0 + jnp.exp(-o))


def _make_mlp(B, DU, DM, DR, NR, KW, H1, BLK):
    grid = (B // BLK,)
    row = lambda i: (i, 0)
    rep = lambda i: (0, 0)
    return pl.pallas_call(
        _mlp_body,
        grid=grid,
        in_specs=[
            pl.BlockSpec((BLK, DU), row),
            pl.BlockSpec((BLK, DM), row),
            pl.BlockSpec((BLK, 1), row),
            pl.BlockSpec((BLK, KW), row),
            pl.BlockSpec((BLK, 1), row),
            pl.BlockSpec((NR, DR), rep),
            pl.BlockSpec((KW, 32), rep),
            pl.BlockSpec((1, 32), rep),
            pl.BlockSpec((DU, H1), rep),
            pl.BlockSpec((DM, H1), rep),
            pl.BlockSpec((DR, H1), rep),
            pl.BlockSpec((32, H1), rep),
            pl.BlockSpec((1, H1), rep),
            pl.BlockSpec((1, H1), rep),
            pl.BlockSpec((H1, 1), rep),
            pl.BlockSpec((1, 1), rep),
        ],
        out_specs=pl.BlockSpec((BLK, 1), row),
        out_shape=jax.ShapeDtypeStruct((B, 1), jnp.float32),
    )


def kernel(user_id, movie_id, region_id, keywords, age, U, M, R, Wk, bk, W1,
           b1, W2, b2):
    B = user_id.shape[0]
    DU, DM, DR = U.shape[1], M.shape[1], R.shape[1]
    NR = R.shape[0]
    KW = keywords.shape[1]
    H1 = W1.shape[1]

    sc_gather = _make_sc_gather(B, DU, DM)
    u, m = sc_gather(user_id, movie_id, U, M)

    mlp = _make_mlp(B, DU, DM, DR, NR, KW, H1, BLK=2048)
    out = mlp(u, m, region_id.reshape(B, 1), keywords, age.reshape(B, 1),
              R, Wk, bk.reshape(1, -1),
              W1[:DU], W1[DU:DU + DM], W1[DU + DM:DU + DM + DR],
              W1[DU + DM + DR:DU + DM + DR + 32], W1[-1:],
              b1.reshape(1, -1), W2, b2.reshape(1, 1))
    return out.reshape(B)
